# baseline (speedup 1.0000x reference)
import functools

import jax
import jax.numpy as jnp
from jax import lax
from jax.experimental import pallas as pl
from jax.experimental.pallas import tpu as pltpu
from jax.experimental.pallas import tpu_sc as plsc

_K = 20
_KP = 24
_EPS = 1e-5
_NC = 2
_NS = 16
_NW = _NC * _NS


def _neg_d2_body(x_rows_ref, x_all_ref, out_ref, *, block_rows):
    i = pl.program_id(1)
    xr = x_rows_ref[0]
    xa = x_all_ref[0]
    sq_r = jnp.sum(xr * xr, axis=-1, keepdims=True)
    sq_a = jnp.sum(xa * xa, axis=-1, keepdims=True).T
    xy = lax.dot_general(xr, xa, (((1,), (1,)), ((), ())),
                         preferred_element_type=jnp.float32)
    nd = jnp.minimum(2.0 * xy - sq_r - sq_a, 0.0)
    rows = lax.broadcasted_iota(jnp.int32, nd.shape, 0) + i * block_rows
    cols = lax.broadcasted_iota(jnp.int32, nd.shape, 1)
    out_ref[0] = jnp.where(rows == cols, 1.0, nd)


def _neg_d2(x, block_rows=256):
    B, N, F = x.shape
    grid = (B, N // block_rows)
    return pl.pallas_call(
        functools.partial(_neg_d2_body, block_rows=block_rows),
        grid=grid,
        in_specs=[
            pl.BlockSpec((1, block_rows, F), lambda b, i: (b, i, 0)),
            pl.BlockSpec((1, N, F), lambda b, i: (b, 0, 0)),
        ],
        out_specs=pl.BlockSpec((1, block_rows, N), lambda b, i: (b, i, 0)),
        out_shape=jax.ShapeDtypeStruct((B, N, N), jnp.float32),
    )(x, x)


def _sc_gather_agg(bv_flat, idx_pad):
    R, C = bv_flat.shape
    rows_per_w = R // _NW
    BLK = 32
    NPAIR = BLK // 2
    NBLK = rows_per_w // BLK
    NCG = C // 16

    mesh = plsc.VectorSubcoreMesh(core_axis_name="c", subcore_axis_name="s",
                                  num_cores=_NC, num_subcores=_NS)
    out_t = tuple(jax.ShapeDtypeStruct((R, C), jnp.float32) for _ in range(4))

    def body(idx_hbm, bv_hbm, s1_hbm, s2_hbm, mx_hbm, mn_hbm,
             idx_v, gat_v, o1, o2, o3, o4, sem0, sem1):
        wid = lax.axis_index("s") * _NC + lax.axis_index("c")
        base = wid * rows_per_w

        def start(r, buf, sem):
            pltpu.async_copy(bv_hbm.at[idx_v.at[r, pl.ds(0, _KP)]],
                             gat_v.at[buf], sem)

        def drain(buf, sem):
            pltpu.make_async_copy(bv_hbm.at[idx_v.at[0, pl.ds(0, _KP)]],
                                  gat_v.at[buf], sem).wait()

        def reduce_row(r, buf):
            for c in range(NCG):
                sl = pl.ds(c * 16, 16)
                v = gat_v[buf, 0, sl]
                s1 = v
                s2 = v * v
                mx = v
                mn = v
                for k in range(1, _K):
                    v = gat_v[buf, k, sl]
                    s1 = s1 + v
                    s2 = s2 + v * v
                    mx = jnp.maximum(mx, v)
                    mn = jnp.minimum(mn, v)
                o1[r, sl] = s1
                o2[r, sl] = s2
                o3[r, sl] = mx
                o4[r, sl] = mn

        def block(blk, _):
            rbase = base + blk * BLK
            pltpu.sync_copy(idx_hbm.at[pl.ds(rbase, BLK)], idx_v)
            start(0, 0, sem0)

            def pair(p, _):
                r0 = 2 * p
                start(r0 + 1, 1, sem1)
                drain(0, sem0)
                reduce_row(r0, 0)

                @pl.when(p < NPAIR - 1)
                def _():
                    start(r0 + 2, 0, sem0)

                drain(1, sem1)
                reduce_row(r0 + 1, 1)
                return 0

            lax.fori_loop(0, NPAIR, pair, 0)
            pltpu.sync_copy(o1, s1_hbm.at[pl.ds(rbase, BLK)])
            pltpu.sync_copy(o2, s2_hbm.at[pl.ds(rbase, BLK)])
            pltpu.sync_copy(o3, mx_hbm.at[pl.ds(rbase, BLK)])
            pltpu.sync_copy(o4, mn_hbm.at[pl.ds(rbase, BLK)])
            return 0

        lax.fori_loop(0, NBLK, block, 0)

    f = pl.kernel(
        body,
        out_type=out_t,
        mesh=mesh,
        compiler_params=pltpu.CompilerParams(use_tc_tiling_on_sc=False),
        scratch_types=[
            pltpu.VMEM((BLK, 32), jnp.int32),
            pltpu.VMEM((2, _KP, C), jnp.float32),
            pltpu.VMEM((BLK, C), jnp.float32),
            pltpu.VMEM((BLK, C), jnp.float32),
            pltpu.VMEM((BLK, C), jnp.float32),
            pltpu.VMEM((BLK, C), jnp.float32),
            pltpu.SemaphoreType.DMA,
            pltpu.SemaphoreType.DMA,
        ],
    )
    return f(idx_pad, bv_flat)


def kernel(x, W, b, gamma, beta):
    B, N, F = x.shape
    R = B * N
    W1 = W[:, :F]
    W2 = W[:, F:]

    nd = _neg_d2(x)
    _, idxk = lax.top_k(nd, _K + 1)
    idx = idxk[:, :, 1:]
    gidx = idx + (jnp.arange(B, dtype=jnp.int32) * N)[:, None, None]
    gidx = gidx.reshape(R, _K)
    gidx = jnp.pad(gidx, ((0, 0), (0, 32 - _K)))

    A = x @ (W1 - W2).T + b
    Bv = x @ W2.T

    S1, S2, M, Mn = _sc_gather_agg(Bv.reshape(R, F), gidx)
    S1 = S1.reshape(B, N, F)
    S2 = S2.reshape(B, N, F)
    M = M.reshape(B, N, F)
    Mn = Mn.reshape(B, N, F)

    invk = 1.0 / _K
    mean = jnp.mean(A + S1 * invk, axis=(0, 1))
    eh2 = jnp.mean(A * A + 2.0 * A * (S1 * invk) + S2 * invk, axis=(0, 1))
    var = eh2 - mean * mean
    scale = gamma * lax.rsqrt(var + _EPS)
    Mx = jnp.where(scale >= 0.0, M, Mn)
    return jax.nn.relu((A + Mx - mean) * scale + beta)


# speedup vs baseline: 4.4503x; 4.4503x over previous
import functools

import jax
import jax.numpy as jnp
from jax import lax
from jax.experimental import pallas as pl
from jax.experimental.pallas import tpu as pltpu
from jax.experimental.pallas import tpu_sc as plsc

_K = 20
_EPS = 1e-5
_NC = 2
_NS = 16
_NW = _NC * _NS
_CHUNK = 16
_NCAND = _K + 4


def _d2_body(x_rows_ref, x_all_ref, d2_ref, cm_ref, *, block_rows):
    i = pl.program_id(1)
    xr = x_rows_ref[0]
    xa = x_all_ref[0]
    sq_r = jnp.sum(xr * xr, axis=-1, keepdims=True)
    sq_a = jnp.sum(xa * xa, axis=-1, keepdims=True).T
    xy = lax.dot_general(xr, xa, (((1,), (1,)), ((), ())),
                         preferred_element_type=jnp.float32)
    d2 = jnp.maximum(sq_r + sq_a - 2.0 * xy, 0.0)
    rows = lax.broadcasted_iota(jnp.int32, d2.shape, 0) + i * block_rows
    cols = lax.broadcasted_iota(jnp.int32, d2.shape, 1)
    d2 = jnp.where(rows == cols, jnp.float32(-1.0), d2)
    d2_ref[0] = d2
    cm_ref[0] = jnp.min(
        d2.reshape(d2.shape[0], d2.shape[1] // _CHUNK, _CHUNK), axis=-1)


def _d2_and_chunkmin(x, block_rows=256):
    B, N, F = x.shape
    grid = (B, N // block_rows)
    return pl.pallas_call(
        functools.partial(_d2_body, block_rows=block_rows),
        grid=grid,
        in_specs=[
            pl.BlockSpec((1, block_rows, F), lambda b, i: (b, i, 0)),
            pl.BlockSpec((1, N, F), lambda b, i: (b, 0, 0)),
        ],
        out_specs=[
            pl.BlockSpec((1, block_rows, N), lambda b, i: (b, i, 0)),
            pl.BlockSpec((1, block_rows, N // _CHUNK), lambda b, i: (b, i, 0)),
        ],
        out_shape=[
            jax.ShapeDtypeStruct((B, N, N), jnp.float32),
            jax.ShapeDtypeStruct((B, N, N // _CHUNK), jnp.float32),
        ],
    )(x, x)


def _merge32(k0, v0, k1, v1, sk, sv):
    rsk = lax.rev(sk, (0,))
    rsv = lax.rev(sv, (0,))
    m = k1 <= rsk
    lo1k = jnp.where(m, k1, rsk)
    lo1v = jnp.where(m, v1, rsv)
    m2 = k0 <= lo1k
    lowk = jnp.where(m2, k0, lo1k)
    lowv = jnp.where(m2, v0, lo1v)
    highk = jnp.where(m2, lo1k, k0)
    highv = jnp.where(m2, lo1v, v0)
    k0, v0 = plsc.sort_key_val(lowk, lowv)
    k1, v1 = plsc.sort_key_val(highk, highv)
    return k0, v0, k1, v1


def _sc_knn_agg(d2c, cm, bv_flat, n):
    R, C = bv_flat.shape
    NCH = cm.shape[1]
    rows_per_w = R // _NW
    BLK = 16
    NBLK = rows_per_w // BLK
    NCG = C // 16

    mesh = plsc.VectorSubcoreMesh(core_axis_name="c", subcore_axis_name="s",
                                  num_cores=_NC, num_subcores=_NS)
    out_t = tuple(jax.ShapeDtypeStruct((R, C), jnp.float32) for _ in range(4))

    def body(d2c_hbm, cm_hbm, bv_hbm, s1_hbm, s2_hbm, mx_hbm, mn_hbm,
             cm_v, cand_v, bvg_v, cidx_v, bidx_v, o1, o2, o3, o4,
             semc, semb):
        wid = lax.axis_index("s") * _NC + lax.axis_index("c")
        base = wid * rows_per_w
        bbase = (base // n) * n

        iota = lax.iota(jnp.int32, 16)

        def select_row(r, lr):
            k0, v0 = plsc.sort_key_val(cm_v[lr, pl.ds(0, 16)], iota)
            k1 = jnp.full((16,), jnp.inf, jnp.float32)
            v1 = iota
            for c in range(1, NCH // 16):
                sk, sv = plsc.sort_key_val(cm_v[lr, pl.ds(c * 16, 16)],
                                           iota + (c * 16))
                k0, v0, k1, v1 = _merge32(k0, v0, k1, v1, sk, sv)
            cbase = r * NCH
            cidx_v[pl.ds(0, 16)] = v0 + cbase
            cidx_v[pl.ds(16, 16)] = v1 + cbase
            ch0 = v0 * _CHUNK
            ch1 = v1 * _CHUNK
            pltpu.async_copy(d2c_hbm.at[cidx_v.at[pl.ds(0, _NCAND)]],
                             cand_v, semc).wait()

            def colbase(j):
                src = ch0 if j < 16 else ch1
                lane = jnp.full((16,), j % 16, jnp.int32)
                return lax.gather(
                    src, lane[:, None],
                    lax.GatherDimensionNumbers(
                        offset_dims=(), collapsed_slice_dims=(0,),
                        start_index_map=(0,)),
                    (1,), mode=lax.GatherScatterMode.PROMISE_IN_BOUNDS)

            k0, v0 = plsc.sort_key_val(cand_v[0, pl.ds(0, 16)],
                                       iota + colbase(0))
            k1 = jnp.full((16,), jnp.inf, jnp.float32)
            v1 = iota
            for j in range(1, _K + 1):
                sk, sv = plsc.sort_key_val(cand_v[j, pl.ds(0, 16)],
                                           iota + colbase(j))
                k0, v0, k1, v1 = _merge32(k0, v0, k1, v1, sk, sv)
            bidx_v[pl.ds(0, 16)] = v0 + bbase
            bidx_v[pl.ds(16, 16)] = v1 + bbase
            pltpu.async_copy(bv_hbm.at[bidx_v.at[pl.ds(0, _NCAND)]],
                             bvg_v, semb).wait()
            for c in range(NCG):
                sl = pl.ds(c * 16, 16)
                v = bvg_v[1, sl]
                s1 = v
                s2 = v * v
                mx = v
                mn = v
                for k in range(2, _K + 1):
                    v = bvg_v[k, sl]
                    s1 = s1 + v
                    s2 = s2 + v * v
                    mx = jnp.maximum(mx, v)
                    mn = jnp.minimum(mn, v)
                o1[lr, sl] = s1
                o2[lr, sl] = s2
                o3[lr, sl] = mx
                o4[lr, sl] = mn

        def block(blk, _):
            rbase = base + blk * BLK
            pltpu.sync_copy(cm_hbm.at[pl.ds(rbase, BLK)], cm_v)

            def row(lr, _):
                select_row(rbase + lr, lr)
                return 0

            lax.fori_loop(0, BLK, row, 0)
            pltpu.sync_copy(o1, s1_hbm.at[pl.ds(rbase, BLK)])
            pltpu.sync_copy(o2, s2_hbm.at[pl.ds(rbase, BLK)])
            pltpu.sync_copy(o3, mx_hbm.at[pl.ds(rbase, BLK)])
            pltpu.sync_copy(o4, mn_hbm.at[pl.ds(rbase, BLK)])
            return 0

        lax.fori_loop(0, NBLK, block, 0)

    f = pl.kernel(
        body,
        out_type=out_t,
        mesh=mesh,
        compiler_params=pltpu.CompilerParams(use_tc_tiling_on_sc=False,
                                             needs_layout_passes=False),
        scratch_types=[
            pltpu.VMEM((BLK, NCH), jnp.float32),
            pltpu.VMEM((_NCAND, _CHUNK), jnp.float32),
            pltpu.VMEM((_NCAND, C), jnp.float32),
            pltpu.VMEM((32,), jnp.int32),
            pltpu.VMEM((32,), jnp.int32),
            pltpu.VMEM((BLK, C), jnp.float32),
            pltpu.VMEM((BLK, C), jnp.float32),
            pltpu.VMEM((BLK, C), jnp.float32),
            pltpu.VMEM((BLK, C), jnp.float32),
            pltpu.SemaphoreType.DMA,
            pltpu.SemaphoreType.DMA,
        ],
    )
    return f(d2c, cm, bv_flat)


def kernel(x, W, b, gamma, beta):
    B, N, F = x.shape
    R = B * N
    W1 = W[:, :F]
    W2 = W[:, F:]

    d2e, cm = _d2_and_chunkmin(x)
    d2c = d2e.reshape(R * (N // _CHUNK), _CHUNK)
    cm = cm.reshape(R, N // _CHUNK)

    A = x @ (W1 - W2).T + b
    Bv = x @ W2.T

    S1, S2, M, Mn = _sc_knn_agg(d2c, cm, Bv.reshape(R, F), N)
    S1 = S1.reshape(B, N, F)
    S2 = S2.reshape(B, N, F)
    M = M.reshape(B, N, F)
    Mn = Mn.reshape(B, N, F)

    invk = 1.0 / _K
    mean = jnp.mean(A + S1 * invk, axis=(0, 1))
    eh2 = jnp.mean(A * A + 2.0 * A * (S1 * invk) + S2 * invk, axis=(0, 1))
    var = eh2 - mean * mean
    scale = gamma * lax.rsqrt(var + _EPS)
    Mx = jnp.where(scale >= 0.0, M, Mn)
    return jax.nn.relu((A + Mx - mean) * scale + beta)


# speedup vs baseline: 5.4171x; 1.2172x over previous
import functools

import jax
import jax.numpy as jnp
from jax import lax
from jax.experimental import pallas as pl
from jax.experimental.pallas import tpu as pltpu
from jax.experimental.pallas import tpu_sc as plsc

_K = 20
_EPS = 1e-5
_NC = 2
_NS = 16
_NW = _NC * _NS
_CHUNK = 16
_NCAND = _K + 4


def _d2_body(x_rows_ref, x_all_ref, d2_ref, cm_ref, *, block_rows):
    i = pl.program_id(1)
    xr = x_rows_ref[0]
    xa = x_all_ref[0]
    sq_r = jnp.sum(xr * xr, axis=-1, keepdims=True)
    sq_a = jnp.sum(xa * xa, axis=-1, keepdims=True).T
    xy = lax.dot_general(xr, xa, (((1,), (1,)), ((), ())),
                         preferred_element_type=jnp.float32)
    d2 = jnp.maximum(sq_r + sq_a - 2.0 * xy, 0.0)
    rows = lax.broadcasted_iota(jnp.int32, d2.shape, 0) + i * block_rows
    cols = lax.broadcasted_iota(jnp.int32, d2.shape, 1)
    d2 = jnp.where(rows == cols, jnp.float32(-1.0), d2)
    d2_ref[0] = d2
    cm_ref[0] = jnp.min(
        d2.reshape(d2.shape[0], d2.shape[1] // _CHUNK, _CHUNK), axis=-1)


def _d2_and_chunkmin(x, block_rows=256):
    B, N, F = x.shape
    grid = (B, N // block_rows)
    return pl.pallas_call(
        functools.partial(_d2_body, block_rows=block_rows),
        grid=grid,
        in_specs=[
            pl.BlockSpec((1, block_rows, F), lambda b, i: (b, i, 0)),
            pl.BlockSpec((1, N, F), lambda b, i: (b, 0, 0)),
        ],
        out_specs=[
            pl.BlockSpec((1, block_rows, N), lambda b, i: (b, i, 0)),
            pl.BlockSpec((1, block_rows, N // _CHUNK), lambda b, i: (b, i, 0)),
        ],
        out_shape=[
            jax.ShapeDtypeStruct((B, N, N), jnp.float32),
            jax.ShapeDtypeStruct((B, N, N // _CHUNK), jnp.float32),
        ],
    )(x, x)


def _merge32(k0, v0, k1, v1, sk, sv):
    rsk = lax.rev(sk, (0,))
    rsv = lax.rev(sv, (0,))
    m = k1 <= rsk
    lo1k = jnp.where(m, k1, rsk)
    lo1v = jnp.where(m, v1, rsv)
    m2 = k0 <= lo1k
    lowk = jnp.where(m2, k0, lo1k)
    lowv = jnp.where(m2, v0, lo1v)
    highk = jnp.where(m2, lo1k, k0)
    highv = jnp.where(m2, lo1v, v0)
    k0, v0 = plsc.sort_key_val(lowk, lowv)
    k1, v1 = plsc.sort_key_val(highk, highv)
    return k0, v0, k1, v1


def _sc_knn_agg(d2c, cm, bv_flat, n):
    R, C = bv_flat.shape
    NCH = cm.shape[1]
    rows_per_w = R // _NW
    BLK = 16
    NBLK = rows_per_w // BLK
    NCG = C // 16

    mesh = plsc.VectorSubcoreMesh(core_axis_name="c", subcore_axis_name="s",
                                  num_cores=_NC, num_subcores=_NS)
    out_t = tuple(jax.ShapeDtypeStruct((R, C), jnp.float32) for _ in range(4))

    def body(d2c_hbm, cm_hbm, bv_hbm, s1_hbm, s2_hbm, mx_hbm, mn_hbm,
             cm_v, cand_v, bvg_v, cidx_v, bidx_v, o1, o2, o3, o4,
             semc, semb):
        wid = lax.axis_index("s") * _NC + lax.axis_index("c")
        base = wid * rows_per_w
        bbase = (base // n) * n

        iota = lax.iota(jnp.int32, 16)

        def phase1(r, lr):
            k0, v0 = plsc.sort_key_val(cm_v[lr, pl.ds(0, 16)], iota)
            k1 = jnp.full((16,), jnp.inf, jnp.float32)
            v1 = iota
            for c in range(1, NCH // 16):
                sk, sv = plsc.sort_key_val(cm_v[lr, pl.ds(c * 16, 16)],
                                           iota + (c * 16))
                k0, v0, k1, v1 = _merge32(k0, v0, k1, v1, sk, sv)
            cbase = r * NCH
            cidx_v[pl.ds(0, 16)] = v0 + cbase
            cidx_v[pl.ds(16, 16)] = v1 + cbase
            pltpu.async_copy(d2c_hbm.at[cidx_v.at[pl.ds(0, _NCAND)]],
                             cand_v, semc)
            return v0 * _CHUNK, v1 * _CHUNK

        def phase2(ch0, ch1):
            def colbase(j):
                src = ch0 if j < 16 else ch1
                lane = jnp.full((16,), j % 16, jnp.int32)
                return lax.gather(
                    src, lane[:, None],
                    lax.GatherDimensionNumbers(
                        offset_dims=(), collapsed_slice_dims=(0,),
                        start_index_map=(0,)),
                    (1,), mode=lax.GatherScatterMode.PROMISE_IN_BOUNDS)

            k0, v0 = plsc.sort_key_val(cand_v[0, pl.ds(0, 16)],
                                       iota + colbase(0))
            k1 = jnp.full((16,), jnp.inf, jnp.float32)
            v1 = iota
            for j in range(1, _K + 1):
                sk, sv = plsc.sort_key_val(cand_v[j, pl.ds(0, 16)],
                                           iota + colbase(j))
                k0, v0, k1, v1 = _merge32(k0, v0, k1, v1, sk, sv)
            bidx_v[pl.ds(0, 16)] = v0 + bbase
            bidx_v[pl.ds(16, 16)] = v1 + bbase
            pltpu.async_copy(bv_hbm.at[bidx_v.at[pl.ds(0, _NCAND)]],
                             bvg_v, semb)

        def drain_chunks():
            pltpu.make_async_copy(d2c_hbm.at[cidx_v.at[pl.ds(0, _NCAND)]],
                                  cand_v, semc).wait()

        def drain_bv():
            pltpu.make_async_copy(bv_hbm.at[bidx_v.at[pl.ds(0, _NCAND)]],
                                  bvg_v, semb).wait()

        def reduce_row(lr):
            for c in range(NCG):
                sl = pl.ds(c * 16, 16)
                v = bvg_v[1, sl]
                s1 = v
                s2 = v * v
                mx = v
                mn = v
                for k in range(2, _K + 1):
                    v = bvg_v[k, sl]
                    s1 = s1 + v
                    s2 = s2 + v * v
                    mx = jnp.maximum(mx, v)
                    mn = jnp.minimum(mn, v)
                o1[lr, sl] = s1
                o2[lr, sl] = s2
                o3[lr, sl] = mx
                o4[lr, sl] = mn

        def block(blk, _):
            rbase = base + blk * BLK
            pltpu.sync_copy(cm_hbm.at[pl.ds(rbase, BLK)], cm_v)

            def row(lr, _):
                ch0, ch1 = phase1(rbase + lr, lr)

                @pl.when(lr > 0)
                def _():
                    drain_bv()
                    reduce_row(lr - 1)

                drain_chunks()
                phase2(ch0, ch1)
                return 0

            lax.fori_loop(0, BLK, row, 0)
            drain_bv()
            reduce_row(BLK - 1)
            pltpu.sync_copy(o1, s1_hbm.at[pl.ds(rbase, BLK)])
            pltpu.sync_copy(o2, s2_hbm.at[pl.ds(rbase, BLK)])
            pltpu.sync_copy(o3, mx_hbm.at[pl.ds(rbase, BLK)])
            pltpu.sync_copy(o4, mn_hbm.at[pl.ds(rbase, BLK)])
            return 0

        lax.fori_loop(0, NBLK, block, 0)

    f = pl.kernel(
        body,
        out_type=out_t,
        mesh=mesh,
        compiler_params=pltpu.CompilerParams(use_tc_tiling_on_sc=False,
                                             needs_layout_passes=False),
        scratch_types=[
            pltpu.VMEM((BLK, NCH), jnp.float32),
            pltpu.VMEM((_NCAND, _CHUNK), jnp.float32),
            pltpu.VMEM((_NCAND, C), jnp.float32),
            pltpu.VMEM((32,), jnp.int32),
            pltpu.VMEM((32,), jnp.int32),
            pltpu.VMEM((BLK, C), jnp.float32),
            pltpu.VMEM((BLK, C), jnp.float32),
            pltpu.VMEM((BLK, C), jnp.float32),
            pltpu.VMEM((BLK, C), jnp.float32),
            pltpu.SemaphoreType.DMA,
            pltpu.SemaphoreType.DMA,
        ],
    )
    return f(d2c, cm, bv_flat)


def kernel(x, W, b, gamma, beta):
    B, N, F = x.shape
    R = B * N
    W1 = W[:, :F]
    W2 = W[:, F:]

    d2e, cm = _d2_and_chunkmin(x)
    d2c = d2e.reshape(R * (N // _CHUNK), _CHUNK)
    cm = cm.reshape(R, N // _CHUNK)

    A = x @ (W1 - W2).T + b
    Bv = x @ W2.T

    S1, S2, M, Mn = _sc_knn_agg(d2c, cm, Bv.reshape(R, F), N)
    S1 = S1.reshape(B, N, F)
    S2 = S2.reshape(B, N, F)
    M = M.reshape(B, N, F)
    Mn = Mn.reshape(B, N, F)

    invk = 1.0 / _K
    mean = jnp.mean(A + S1 * invk, axis=(0, 1))
    eh2 = jnp.mean(A * A + 2.0 * A * (S1 * invk) + S2 * invk, axis=(0, 1))
    var = eh2 - mean * mean
    scale = gamma * lax.rsqrt(var + _EPS)
    Mx = jnp.where(scale >= 0.0, M, Mn)
    return jax.nn.relu((A + Mx - mean) * scale + beta)


# speedup vs baseline: 7.6019x; 1.4033x over previous
import functools

import jax
import jax.numpy as jnp
from jax import lax
from jax.experimental import pallas as pl
from jax.experimental.pallas import tpu as pltpu
from jax.experimental.pallas import tpu_sc as plsc

_K = 20
_EPS = 1e-5
_NC = 2
_NS = 16
_NW = _NC * _NS
_NCH = 128
_NCAND = _K + 4


def _d2_body(x_rows_ref, x_all_ref, d2_ref, cm_ref, *, block_rows):
    i = pl.program_id(1)
    xr = x_rows_ref[0]
    xa = x_all_ref[0]
    sq_r = jnp.sum(xr * xr, axis=-1, keepdims=True)
    sq_a = jnp.sum(xa * xa, axis=-1, keepdims=True).T
    xy = lax.dot_general(xr, xa, (((1,), (1,)), ((), ())),
                         preferred_element_type=jnp.float32)
    d2 = jnp.maximum(sq_r + sq_a - 2.0 * xy, 0.0)
    rows = lax.broadcasted_iota(jnp.int32, d2.shape, 0) + i * block_rows
    cols = lax.broadcasted_iota(jnp.int32, d2.shape, 1)
    d2 = jnp.where(rows == cols, jnp.float32(-1.0), d2)
    d2_ref[0] = d2
    cm_ref[0] = jnp.min(
        d2.reshape(d2.shape[0], d2.shape[1] // _NCH, _NCH), axis=1)


def _d2_and_chunkmin(x, block_rows=256):
    B, N, F = x.shape
    grid = (B, N // block_rows)
    return pl.pallas_call(
        functools.partial(_d2_body, block_rows=block_rows),
        grid=grid,
        in_specs=[
            pl.BlockSpec((1, block_rows, F), lambda b, i: (b, i, 0)),
            pl.BlockSpec((1, N, F), lambda b, i: (b, 0, 0)),
        ],
        out_specs=[
            pl.BlockSpec((1, block_rows, N), lambda b, i: (b, i, 0)),
            pl.BlockSpec((1, block_rows, _NCH), lambda b, i: (b, i, 0)),
        ],
        out_shape=[
            jax.ShapeDtypeStruct((B, N, N), jnp.float32),
            jax.ShapeDtypeStruct((B, N, _NCH), jnp.float32),
        ],
    )(x, x)


def _merge32(k0, v0, k1, v1, sk, sv):
    rsk = lax.rev(sk, (0,))
    rsv = lax.rev(sv, (0,))
    m = k1 <= rsk
    lo1k = jnp.where(m, k1, rsk)
    lo1v = jnp.where(m, v1, rsv)
    m2 = k0 <= lo1k
    lowk = jnp.where(m2, k0, lo1k)
    lowv = jnp.where(m2, v0, lo1v)
    highk = jnp.where(m2, lo1k, k0)
    highv = jnp.where(m2, lo1v, v0)
    k0, v0 = plsc.sort_key_val(lowk, lowv)
    k1, v1 = plsc.sort_key_val(highk, highv)
    return k0, v0, k1, v1


def _lane_bcast(src, j):
    lane = jnp.full((16,), j, jnp.int32)
    return lax.gather(
        src, lane[:, None],
        lax.GatherDimensionNumbers(
            offset_dims=(), collapsed_slice_dims=(0,), start_index_map=(0,)),
        (1,), mode=lax.GatherScatterMode.PROMISE_IN_BOUNDS)


def _sc_knn_agg(d2, cm, bv_flat, n):
    R, C = bv_flat.shape
    N = d2.shape[1]
    rows_per_w = R // _NW
    BLK = 16
    NBLK = rows_per_w // BLK
    NCG = C // 16

    mesh = plsc.VectorSubcoreMesh(core_axis_name="c", subcore_axis_name="s",
                                  num_cores=_NC, num_subcores=_NS)
    out_t = tuple(jax.ShapeDtypeStruct((R, C), jnp.float32) for _ in range(4))

    def body(d2_hbm, cm_hbm, bv_hbm, s1_hbm, s2_hbm, mx_hbm, mn_hbm,
             d2_v, cm_v, bvg_v, bidx_v, o1, o2, o3, o4,
             semd0, semd1, semb):
        wid = lax.axis_index("s") * _NC + lax.axis_index("c")
        base = wid * rows_per_w
        bbase = (base // n) * n

        iota = lax.iota(jnp.int32, 16)
        giota = iota * _NCH

        def start_d2(blk, half):
            pltpu.async_copy(d2_hbm.at[pl.ds(base + blk * BLK, BLK)],
                             d2_v.at[half], semd0 if half == 0 else semd1)

        def drain_d2(half):
            pltpu.make_async_copy(d2_hbm.at[pl.ds(base, BLK)],
                                  d2_v.at[half],
                                  semd0 if half == 0 else semd1).wait()

        def drain_bv():
            pltpu.make_async_copy(bv_hbm.at[bidx_v.at[pl.ds(0, _NCAND)]],
                                  bvg_v, semb).wait()

        def phase1(lr):
            k0, v0 = plsc.sort_key_val(cm_v[lr, pl.ds(0, 16)], iota)
            k1 = jnp.full((16,), jnp.inf, jnp.float32)
            v1 = iota
            for c in range(1, _NCH // 16):
                sk, sv = plsc.sort_key_val(cm_v[lr, pl.ds(c * 16, 16)],
                                           iota + (c * 16))
                k0, v0, k1, v1 = _merge32(k0, v0, k1, v1, sk, sv)
            return v0, v1

        def phase2(half, lr, c0, c1):
            rowv = jnp.broadcast_to(lr, (16,)).astype(jnp.int32)
            cols0 = _lane_bcast(c0, 0) + giota
            vals = plsc.load_gather(d2_v.at[half], [rowv, cols0])
            k0, v0 = plsc.sort_key_val(vals, cols0)
            k1 = jnp.full((16,), jnp.inf, jnp.float32)
            v1 = iota
            for j in range(1, _K + 1):
                src = c0 if j < 16 else c1
                cols = _lane_bcast(src, j % 16) + giota
                vals = plsc.load_gather(d2_v.at[half], [rowv, cols])
                sk, sv = plsc.sort_key_val(vals, cols)
                k0, v0, k1, v1 = _merge32(k0, v0, k1, v1, sk, sv)
            bidx_v[pl.ds(0, 16)] = v0 + bbase
            bidx_v[pl.ds(16, 16)] = v1 + bbase
            pltpu.async_copy(bv_hbm.at[bidx_v.at[pl.ds(0, _NCAND)]],
                             bvg_v, semb)

        def reduce_row(lr):
            for c in range(NCG):
                sl = pl.ds(c * 16, 16)
                v = bvg_v[1, sl]
                s1 = v
                s2 = v * v
                mx = v
                mn = v
                for k in range(2, _K + 1):
                    v = bvg_v[k, sl]
                    s1 = s1 + v
                    s2 = s2 + v * v
                    mx = jnp.maximum(mx, v)
                    mn = jnp.minimum(mn, v)
                o1[lr, sl] = s1
                o2[lr, sl] = s2
                o3[lr, sl] = mx
                o4[lr, sl] = mn

        def do_block(blk, half):
            rbase = base + blk * BLK
            pltpu.sync_copy(cm_hbm.at[pl.ds(rbase, BLK)], cm_v)
            drain_d2(half)

            def row(lr, _):
                c0, c1 = phase1(lr)

                @pl.when(lr > 0)
                def _():
                    drain_bv()
                    reduce_row(lr - 1)

                phase2(half, lr, c0, c1)
                return 0

            lax.fori_loop(0, BLK, row, 0)
            @pl.when(blk + 2 < NBLK)
            def _():
                start_d2(blk + 2, half)

            drain_bv()
            reduce_row(BLK - 1)
            pltpu.sync_copy(o1, s1_hbm.at[pl.ds(rbase, BLK)])
            pltpu.sync_copy(o2, s2_hbm.at[pl.ds(rbase, BLK)])
            pltpu.sync_copy(o3, mx_hbm.at[pl.ds(rbase, BLK)])
            pltpu.sync_copy(o4, mn_hbm.at[pl.ds(rbase, BLK)])

        start_d2(0, 0)
        start_d2(1, 1)

        def pair(q, _):
            blk = 2 * q
            do_block(blk, 0)
            do_block(blk + 1, 1)
            return 0

        lax.fori_loop(0, NBLK // 2, pair, 0)

    f = pl.kernel(
        body,
        out_type=out_t,
        mesh=mesh,
        compiler_params=pltpu.CompilerParams(use_tc_tiling_on_sc=False,
                                             needs_layout_passes=False),
        scratch_types=[
            pltpu.VMEM((2, BLK, N), jnp.float32),
            pltpu.VMEM((BLK, _NCH), jnp.float32),
            pltpu.VMEM((_NCAND, C), jnp.float32),
            pltpu.VMEM((32,), jnp.int32),
            pltpu.VMEM((BLK, C), jnp.float32),
            pltpu.VMEM((BLK, C), jnp.float32),
            pltpu.VMEM((BLK, C), jnp.float32),
            pltpu.VMEM((BLK, C), jnp.float32),
            pltpu.SemaphoreType.DMA,
            pltpu.SemaphoreType.DMA,
            pltpu.SemaphoreType.DMA,
        ],
    )
    return f(d2, cm, bv_flat)


def kernel(x, W, b, gamma, beta):
    B, N, F = x.shape
    R = B * N
    W1 = W[:, :F]
    W2 = W[:, F:]

    d2, cm = _d2_and_chunkmin(x)
    d2 = d2.reshape(R, N)
    cm = cm.reshape(R, _NCH)

    A = x @ (W1 - W2).T + b
    Bv = x @ W2.T

    S1, S2, M, Mn = _sc_knn_agg(d2, cm, Bv.reshape(R, F), N)
    S1 = S1.reshape(B, N, F)
    S2 = S2.reshape(B, N, F)
    M = M.reshape(B, N, F)
    Mn = Mn.reshape(B, N, F)

    invk = 1.0 / _K
    mean = jnp.mean(A + S1 * invk, axis=(0, 1))
    eh2 = jnp.mean(A * A + 2.0 * A * (S1 * invk) + S2 * invk, axis=(0, 1))
    var = eh2 - mean * mean
    scale = gamma * lax.rsqrt(var + _EPS)
    Mx = jnp.where(scale >= 0.0, M, Mn)
    return jax.nn.relu((A + Mx - mean) * scale + beta)


# speedup vs baseline: 7.8081x; 1.0271x over previous
import functools

import jax
import jax.numpy as jnp
from jax import lax
from jax.experimental import pallas as pl
from jax.experimental.pallas import tpu as pltpu
from jax.experimental.pallas import tpu_sc as plsc

_K = 20
_EPS = 1e-5
_NC = 2
_NS = 16
_NW = _NC * _NS
_NCH = 128
_NCAND = _K + 4


def _d2_body(x_rows_ref, x_all_ref, w_ref, b_ref, d2_ref, cm_ref,
             a_ref, bv_ref, *, block_rows):
    i = pl.program_id(1)
    xr = x_rows_ref[0]
    xa = x_all_ref[0]
    sq_r = jnp.sum(xr * xr, axis=-1, keepdims=True)
    sq_a = jnp.sum(xa * xa, axis=-1, keepdims=True).T
    xy = lax.dot_general(xr, xa, (((1,), (1,)), ((), ())),
                         preferred_element_type=jnp.float32)
    d2 = jnp.maximum(sq_r + sq_a - 2.0 * xy, 0.0)
    rows = lax.broadcasted_iota(jnp.int32, d2.shape, 0) + i * block_rows
    cols = lax.broadcasted_iota(jnp.int32, d2.shape, 1)
    d2 = jnp.where(rows == cols, jnp.float32(-1.0), d2)
    d2_ref[0] = d2
    cm_ref[0] = jnp.min(
        d2.reshape(d2.shape[0], d2.shape[1] // _NCH, _NCH), axis=1)
    F = xr.shape[1]
    w1 = w_ref[:, :F]
    w2 = w_ref[:, F:]
    a_ref[0] = lax.dot_general(xr, w1 - w2, (((1,), (1,)), ((), ())),
                               preferred_element_type=jnp.float32) + b_ref[0]
    bv_ref[0] = lax.dot_general(xr, w2, (((1,), (1,)), ((), ())),
                                preferred_element_type=jnp.float32)


def _d2_and_chunkmin(x, W, bias, block_rows=256):
    B, N, F = x.shape
    O = W.shape[0]
    grid = (B, N // block_rows)
    return pl.pallas_call(
        functools.partial(_d2_body, block_rows=block_rows),
        grid=grid,
        in_specs=[
            pl.BlockSpec((1, block_rows, F), lambda b, i: (b, i, 0)),
            pl.BlockSpec((1, N, F), lambda b, i: (b, 0, 0)),
            pl.BlockSpec((O, 2 * F), lambda b, i: (0, 0)),
            pl.BlockSpec((1, O), lambda b, i: (0, 0)),
        ],
        out_specs=[
            pl.BlockSpec((1, block_rows, N), lambda b, i: (b, i, 0)),
            pl.BlockSpec((1, block_rows, _NCH), lambda b, i: (b, i, 0)),
            pl.BlockSpec((1, block_rows, O), lambda b, i: (b, i, 0)),
            pl.BlockSpec((1, block_rows, O), lambda b, i: (b, i, 0)),
        ],
        out_shape=[
            jax.ShapeDtypeStruct((B, N, N), jnp.float32),
            jax.ShapeDtypeStruct((B, N, _NCH), jnp.float32),
            jax.ShapeDtypeStruct((B, N, O), jnp.float32),
            jax.ShapeDtypeStruct((B, N, O), jnp.float32),
        ],
    )(x, x, W, bias.reshape(1, O))


def _stats_body(a_ref, s1_ref, s2_ref, p1_ref, p2_ref):
    i = pl.program_id(0)
    a = a_ref[...]
    s1 = s1_ref[...] * (1.0 / _K)
    s2 = s2_ref[...] * (1.0 / _K)
    p1 = jnp.sum(a + s1, axis=0, keepdims=True)
    p2 = jnp.sum(a * a + 2.0 * a * s1 + s2, axis=0, keepdims=True)

    @pl.when(i == 0)
    def _():
        p1_ref[...] = jnp.zeros_like(p1_ref)
        p2_ref[...] = jnp.zeros_like(p2_ref)

    p1_ref[...] += p1
    p2_ref[...] += p2


def _bn_stats(A, S1, S2, block_rows=2048):
    R, O = A.shape
    grid = (R // block_rows,)
    return pl.pallas_call(
        _stats_body,
        grid=grid,
        in_specs=[pl.BlockSpec((block_rows, O), lambda i: (i, 0))] * 3,
        out_specs=[pl.BlockSpec((1, O), lambda i: (0, 0))] * 2,
        out_shape=[jax.ShapeDtypeStruct((1, O), jnp.float32)] * 2,
    )(A, S1, S2)


def _norm_body(a_ref, mx_ref, mn_ref, p1_ref, p2_ref, g_ref, bt_ref,
               o_ref, *, rows_total):
    mean = p1_ref[...] * (1.0 / rows_total)
    var = p2_ref[...] * (1.0 / rows_total) - mean * mean
    scale = g_ref[...] * lax.rsqrt(var + _EPS)
    mxx = jnp.where(scale >= 0.0, mx_ref[...], mn_ref[...])
    o_ref[...] = jnp.maximum(
        (a_ref[...] + mxx - mean) * scale + bt_ref[...], 0.0)


def _bn_normalize(A, Mx, Mn, p1, p2, gamma, beta, block_rows=2048):
    R, O = A.shape
    grid = (R // block_rows,)
    return pl.pallas_call(
        functools.partial(_norm_body, rows_total=R),
        grid=grid,
        in_specs=[
            pl.BlockSpec((block_rows, O), lambda i: (i, 0)),
            pl.BlockSpec((block_rows, O), lambda i: (i, 0)),
            pl.BlockSpec((block_rows, O), lambda i: (i, 0)),
            pl.BlockSpec((1, O), lambda i: (0, 0)),
            pl.BlockSpec((1, O), lambda i: (0, 0)),
            pl.BlockSpec((1, O), lambda i: (0, 0)),
            pl.BlockSpec((1, O), lambda i: (0, 0)),
        ],
        out_specs=pl.BlockSpec((block_rows, O), lambda i: (i, 0)),
        out_shape=jax.ShapeDtypeStruct((R, O), jnp.float32),
    )(A, Mx, Mn, p1, p2, gamma.reshape(1, O), beta.reshape(1, O))


def _merge32(k0, v0, k1, v1, sk, sv):
    rsk = lax.rev(sk, (0,))
    rsv = lax.rev(sv, (0,))
    m = k1 <= rsk
    lo1k = jnp.where(m, k1, rsk)
    lo1v = jnp.where(m, v1, rsv)
    m2 = k0 <= lo1k
    lowk = jnp.where(m2, k0, lo1k)
    lowv = jnp.where(m2, v0, lo1v)
    highk = jnp.where(m2, lo1k, k0)
    highv = jnp.where(m2, lo1v, v0)
    k0, v0 = plsc.sort_key_val(lowk, lowv)
    k1, v1 = plsc.sort_key_val(highk, highv)
    return k0, v0, k1, v1


def _lane_bcast(src, j):
    lane = jnp.full((16,), j, jnp.int32)
    return lax.gather(
        src, lane[:, None],
        lax.GatherDimensionNumbers(
            offset_dims=(), collapsed_slice_dims=(0,), start_index_map=(0,)),
        (1,), mode=lax.GatherScatterMode.PROMISE_IN_BOUNDS)


def _sc_knn_agg(d2, cm, bv_flat, n):
    R, C = bv_flat.shape
    N = d2.shape[1]
    rows_per_w = R // _NW
    BLK = 16
    NBLK = rows_per_w // BLK
    NCG = C // 16

    mesh = plsc.VectorSubcoreMesh(core_axis_name="c", subcore_axis_name="s",
                                  num_cores=_NC, num_subcores=_NS)
    out_t = tuple(jax.ShapeDtypeStruct((R, C), jnp.float32) for _ in range(4))

    def body(d2_hbm, cm_hbm, bv_hbm, s1_hbm, s2_hbm, mx_hbm, mn_hbm,
             d2_v, cm_v, bvg_v, bidx_v, o1, o2, o3, o4,
             semd0, semd1, semb):
        wid = lax.axis_index("s") * _NC + lax.axis_index("c")
        base = wid * rows_per_w
        bbase = (base // n) * n

        iota = lax.iota(jnp.int32, 16)
        giota = iota * _NCH

        def start_d2(blk, half):
            pltpu.async_copy(d2_hbm.at[pl.ds(base + blk * BLK, BLK)],
                             d2_v.at[half], semd0 if half == 0 else semd1)

        def drain_d2(half):
            pltpu.make_async_copy(d2_hbm.at[pl.ds(base, BLK)],
                                  d2_v.at[half],
                                  semd0 if half == 0 else semd1).wait()

        def drain_bv():
            pltpu.make_async_copy(bv_hbm.at[bidx_v.at[pl.ds(0, _NCAND)]],
                                  bvg_v, semb).wait()

        def phase1(lr):
            k0, v0 = plsc.sort_key_val(cm_v[lr, pl.ds(0, 16)], iota)
            k1 = jnp.full((16,), jnp.inf, jnp.float32)
            v1 = iota
            for c in range(1, _NCH // 16):
                sk, sv = plsc.sort_key_val(cm_v[lr, pl.ds(c * 16, 16)],
                                           iota + (c * 16))
                k0, v0, k1, v1 = _merge32(k0, v0, k1, v1, sk, sv)
            return v0, v1

        def phase2(half, lr, c0, c1):
            rowv = jnp.broadcast_to(lr, (16,)).astype(jnp.int32)
            cols0 = _lane_bcast(c0, 0) + giota
            vals = plsc.load_gather(d2_v.at[half], [rowv, cols0])
            k0, v0 = plsc.sort_key_val(vals, cols0)
            k1 = jnp.full((16,), jnp.inf, jnp.float32)
            v1 = iota
            for j in range(1, _K + 1):
                src = c0 if j < 16 else c1
                cols = _lane_bcast(src, j % 16) + giota
                vals = plsc.load_gather(d2_v.at[half], [rowv, cols])
                sk, sv = plsc.sort_key_val(vals, cols)
                k0, v0, k1, v1 = _merge32(k0, v0, k1, v1, sk, sv)
            bidx_v[pl.ds(0, 16)] = v0 + bbase
            bidx_v[pl.ds(16, 16)] = v1 + bbase
            pltpu.async_copy(bv_hbm.at[bidx_v.at[pl.ds(0, _NCAND)]],
                             bvg_v, semb)

        def reduce_row(lr):
            for c in range(NCG):
                sl = pl.ds(c * 16, 16)
                v = bvg_v[1, sl]
                s1 = v
                s2 = v * v
                mx = v
                mn = v
                for k in range(2, _K + 1):
                    v = bvg_v[k, sl]
                    s1 = s1 + v
                    s2 = s2 + v * v
                    mx = jnp.maximum(mx, v)
                    mn = jnp.minimum(mn, v)
                o1[lr, sl] = s1
                o2[lr, sl] = s2
                o3[lr, sl] = mx
                o4[lr, sl] = mn

        def do_block(blk, half):
            rbase = base + blk * BLK
            pltpu.sync_copy(cm_hbm.at[pl.ds(rbase, BLK)], cm_v)
            drain_d2(half)

            def row(lr, _):
                c0, c1 = phase1(lr)

                @pl.when(lr > 0)
                def _():
                    drain_bv()
                    reduce_row(lr - 1)

                phase2(half, lr, c0, c1)
                return 0

            lax.fori_loop(0, BLK, row, 0)
            @pl.when(blk + 2 < NBLK)
            def _():
                start_d2(blk + 2, half)

            drain_bv()
            reduce_row(BLK - 1)
            pltpu.sync_copy(o1, s1_hbm.at[pl.ds(rbase, BLK)])
            pltpu.sync_copy(o2, s2_hbm.at[pl.ds(rbase, BLK)])
            pltpu.sync_copy(o3, mx_hbm.at[pl.ds(rbase, BLK)])
            pltpu.sync_copy(o4, mn_hbm.at[pl.ds(rbase, BLK)])

        start_d2(0, 0)
        start_d2(1, 1)

        def pair(q, _):
            blk = 2 * q
            do_block(blk, 0)
            do_block(blk + 1, 1)
            return 0

        lax.fori_loop(0, NBLK // 2, pair, 0)

    f = pl.kernel(
        body,
        out_type=out_t,
        mesh=mesh,
        compiler_params=pltpu.CompilerParams(use_tc_tiling_on_sc=False,
                                             needs_layout_passes=False),
        scratch_types=[
            pltpu.VMEM((2, BLK, N), jnp.float32),
            pltpu.VMEM((BLK, _NCH), jnp.float32),
            pltpu.VMEM((_NCAND, C), jnp.float32),
            pltpu.VMEM((32,), jnp.int32),
            pltpu.VMEM((BLK, C), jnp.float32),
            pltpu.VMEM((BLK, C), jnp.float32),
            pltpu.VMEM((BLK, C), jnp.float32),
            pltpu.VMEM((BLK, C), jnp.float32),
            pltpu.SemaphoreType.DMA,
            pltpu.SemaphoreType.DMA,
            pltpu.SemaphoreType.DMA,
        ],
    )
    return f(d2, cm, bv_flat)


def kernel(x, W, b, gamma, beta):
    B, N, F = x.shape
    R = B * N

    d2, cm, A, Bv = _d2_and_chunkmin(x, W, b)
    d2 = d2.reshape(R, N)
    cm = cm.reshape(R, _NCH)
    A = A.reshape(R, F)
    Bv = Bv.reshape(R, F)

    S1, S2, M, Mn = _sc_knn_agg(d2, cm, Bv, N)
    p1, p2 = _bn_stats(A, S1, S2)
    out = _bn_normalize(A, M, Mn, p1, p2, gamma, beta)
    return out.reshape(B, N, F)
